# Initial kernel scaffold; baseline (speedup 1.0000x reference)
import jax
import jax.numpy as jnp
from jax.experimental import pallas as pl


def kernel(x, edge_index, edge_types, edge_emb, W_self, b_self, W_msg, b_msg, ln_gamma, ln_beta):
    raise NotImplementedError("write your pallas kernel here")


# speedup vs baseline: 1.6096x; 1.6096x over previous
import jax
import jax.numpy as jnp
from jax import lax
from jax.experimental import pallas as pl
from jax.experimental.pallas import tpu as pltpu
from jax.experimental.pallas import tpu_sc as plsc

N = 10000
D = 128
E = 320000
T = 8

NC = 2
NS = 16
NW = NC * NS
EPT = E // NW
K = 80
NSUP = 25
NSUPS = EPT // (K * NSUP)
RPT = 624
TAIL = N - NS * RPT


def _sc_body(x_hbm, emb_hbm, src_hbm, dst_hbm, typ_hbm, zb_hbm,
             agg_hbm,
             src_v, dst_v, typ_v, rows_v, acc_sh, sem):
    c = lax.axis_index("c")
    s = lax.axis_index("s")
    wid = c * NS + s

    r0 = s * RPT
    pltpu.sync_copy(zb_hbm.at[pl.ds(r0, RPT)], acc_sh.at[pl.ds(r0, RPT)])

    @pl.when(s == NS - 1)
    def _init_tail():
        t0 = NS * RPT
        pltpu.sync_copy(zb_hbm.at[pl.ds(t0, TAIL)], acc_sh.at[pl.ds(t0, TAIL)])

    plsc.subcore_barrier()

    def superchunk(u, carry):
        pltpu.sync_copy(src_hbm.at[wid, u], src_v)
        pltpu.sync_copy(dst_hbm.at[wid, u], dst_v)
        pltpu.sync_copy(typ_hbm.at[wid, u], typ_v)

        def chunk(j, carry2):
            pltpu.async_copy(x_hbm.at[src_v.at[j]], rows_v, sem).wait()
            pltpu.async_copy(emb_hbm.at[typ_v.at[j]], rows_v, sem,
                             add=True).wait()
            pltpu.sync_copy(rows_v, acc_sh.at[dst_v.at[j]], add=True)
            return carry2

        return lax.fori_loop(0, NSUP, chunk, carry)

    lax.fori_loop(0, NSUPS, superchunk, 0)

    plsc.subcore_barrier()

    pltpu.sync_copy(acc_sh.at[pl.ds(r0, RPT)], agg_hbm.at[c, pl.ds(r0, RPT)])

    @pl.when(s == NS - 1)
    def _write_tail():
        t0 = NS * RPT
        pltpu.sync_copy(acc_sh.at[pl.ds(t0, TAIL)],
                        agg_hbm.at[c, pl.ds(t0, TAIL)])


_sc_aggregate = pl.kernel(
    _sc_body,
    out_type=jax.ShapeDtypeStruct((NC, N, D), jnp.float32),
    mesh=plsc.VectorSubcoreMesh(
        core_axis_name="c", subcore_axis_name="s",
        num_cores=NC, num_subcores=NS,
    ),
    scratch_types=[
        pltpu.VMEM((NSUP, K), jnp.int32),
        pltpu.VMEM((NSUP, K), jnp.int32),
        pltpu.VMEM((NSUP, K), jnp.int32),
        pltpu.VMEM((K, D), jnp.float32),
        pltpu.VMEM_SHARED((N, D), jnp.float32),
        pltpu.SemaphoreType.DMA,
    ],
)


def _tc_body(x_ref, a0_ref, a1_ref, wst_ref, wmt_ref, bias_ref,
             g_ref, b_ref, o_ref):
    m = a0_ref[...] + a1_ref[...]
    h = (jnp.dot(x_ref[...], wst_ref[...], preferred_element_type=jnp.float32)
         + jnp.dot(m, wmt_ref[...], preferred_element_type=jnp.float32)
         + bias_ref[...])
    h = jnp.maximum(h, 0.0)
    mu = jnp.mean(h, axis=-1, keepdims=True)
    var = jnp.mean((h - mu) * (h - mu), axis=-1, keepdims=True)
    hn = (h - mu) * lax.rsqrt(var + 1e-5)
    o_ref[...] = hn * g_ref[...] + b_ref[...]


_R = 400

_tc_epilogue = pl.pallas_call(
    _tc_body,
    grid=(N // _R,),
    in_specs=[
        pl.BlockSpec((_R, D), lambda i: (i, 0)),
        pl.BlockSpec((_R, D), lambda i: (i, 0)),
        pl.BlockSpec((_R, D), lambda i: (i, 0)),
        pl.BlockSpec((D, D), lambda i: (0, 0)),
        pl.BlockSpec((D, D), lambda i: (0, 0)),
        pl.BlockSpec((1, D), lambda i: (0, 0)),
        pl.BlockSpec((1, D), lambda i: (0, 0)),
        pl.BlockSpec((1, D), lambda i: (0, 0)),
    ],
    out_specs=pl.BlockSpec((_R, D), lambda i: (i, 0)),
    out_shape=jax.ShapeDtypeStruct((N, D), jnp.float32),
)


def kernel(x, edge_index, edge_types, edge_emb, W_self, b_self, W_msg, b_msg,
           ln_gamma, ln_beta):
    src = edge_index[0].reshape(NW, NSUPS, NSUP, K).astype(jnp.int32)
    dst = edge_index[1].reshape(NW, NSUPS, NSUP, K).astype(jnp.int32)
    typ = edge_types.reshape(NW, NSUPS, NSUP, K).astype(jnp.int32)
    zb = jnp.zeros((N, D), jnp.float32)

    agg = _sc_aggregate(x, edge_emb, src, dst, typ, zb)

    bias = (b_self + b_msg).reshape(1, D)
    return _tc_epilogue(
        x, agg[0], agg[1],
        W_self.T, W_msg.T, bias,
        ln_gamma.reshape(1, D), ln_beta.reshape(1, D),
    )
